# Initial kernel scaffold; baseline (speedup 1.0000x reference)
import jax
import jax.numpy as jnp
from jax.experimental import pallas as pl


def kernel(token_ids, embedding_table):
    raise NotImplementedError("write your pallas kernel here")


# speedup vs baseline: 1.3481x; 1.3481x over previous
import functools

import jax
import jax.numpy as jnp
from jax.experimental import pallas as pl
from jax.experimental.pallas import tpu as pltpu
from jax.experimental.pallas import tpu_sc as plsc

_WINDOW = 128


def kernel(token_ids, embedding_table):
    B, L = token_ids.shape
    D = embedding_table.shape[1]
    n = B * L
    idx = token_ids.reshape(1, n)
    mesh = plsc.VectorSubcoreMesh(core_axis_name="c", subcore_axis_name="s")

    @functools.partial(
        pl.kernel,
        mesh=mesh,
        out_type=jax.ShapeDtypeStruct((n, D), embedding_table.dtype),
        compiler_params=pltpu.CompilerParams(use_tc_tiling_on_sc=False),
    )
    def gather_kernel(table_hbm, idx_hbm, out_hbm):
        def body(i_vmem, o_vmem):
            pltpu.sync_copy(table_hbm.at[i_vmem.at[0]], o_vmem)

        pltpu.emit_pipeline(
            body,
            grid=(n // _WINDOW,),
            in_specs=[pl.BlockSpec((1, _WINDOW), index_map=lambda i: (0, i))],
            out_specs=[pl.BlockSpec((_WINDOW, D), index_map=lambda i: (i, 0))],
            core_axis_name=("c", "s"),
            dimension_semantics=(pltpu.PARALLEL,),
        )(idx_hbm, out_hbm)

    out = gather_kernel(embedding_table, idx)
    return out.reshape(B, L, D)


# speedup vs baseline: 1.3545x; 1.0047x over previous
import functools

import jax
import jax.numpy as jnp
from jax.experimental import pallas as pl
from jax.experimental.pallas import tpu as pltpu
from jax.experimental.pallas import tpu_sc as plsc

_WINDOW = 128
_K = 8


def kernel(token_ids, embedding_table):
    B, L = token_ids.shape
    D = embedding_table.shape[1]
    n = B * L
    idx = token_ids.reshape(n // _WINDOW, _WINDOW)
    mesh = plsc.VectorSubcoreMesh(core_axis_name="c", subcore_axis_name="s")

    @functools.partial(
        pl.kernel,
        mesh=mesh,
        out_type=jax.ShapeDtypeStruct((n, D), embedding_table.dtype),
        compiler_params=pltpu.CompilerParams(use_tc_tiling_on_sc=False),
    )
    def gather_kernel(table_hbm, idx_hbm, out_hbm):
        def body(i_vmem, o_vmem):
            for j in range(_K):
                pltpu.sync_copy(
                    table_hbm.at[i_vmem.at[j]],
                    o_vmem.at[pl.ds(j * _WINDOW, _WINDOW)],
                )

        pltpu.emit_pipeline(
            body,
            grid=(n // (_K * _WINDOW),),
            in_specs=[pl.BlockSpec((_K, _WINDOW), index_map=lambda i: (i, 0))],
            out_specs=[pl.BlockSpec((_K * _WINDOW, D), index_map=lambda i: (i, 0))],
            core_axis_name=("c", "s"),
            dimension_semantics=(pltpu.PARALLEL,),
        )(idx_hbm, out_hbm)

    out = gather_kernel(embedding_table, idx)
    return out.reshape(B, L, D)
